# baseline (speedup 1.0000x reference)
import functools

import jax
import jax.numpy as jnp
from jax.experimental import pallas as pl
from jax.experimental.pallas import tpu as pltpu

_NEG_SLOPE = 0.01
_K = 16


def _leaky(x):
    return jnp.where(x > 0, x, _NEG_SLOPE * x)


def _fused_kernel(xq_ref, xall_ref, pts_ref, ptile_ref, wcat_ref, bcat_ref,
                  wf_ref, bf_ref, o_ref, *, k, nblk1):
    xall = xall_ref[0]
    xq = xq_ref[0]

    nj = jnp.sum(xall * xall, axis=1, keepdims=True)
    ni = jnp.sum(xq * xq, axis=0, keepdims=True)
    ip2 = jnp.dot(xall * -2.0, xq,
                  precision=jax.lax.Precision.HIGHEST,
                  preferred_element_type=jnp.float32)
    d = (ip2 + nj) + ni

    neg_inf = jnp.float32(-jnp.inf)
    n_all = d.shape[0]

    def _desc_maxima(dq, count):
        ms = [jnp.max(dq, axis=0, keepdims=True)]
        for _ in range(count - 1):
            ms.append(jnp.max(jnp.where(dq >= ms[-1], neg_inf, dq),
                              axis=0, keepdims=True))
        return ms

    ka = k + 1
    a = _desc_maxima(d[: n_all // 2], ka)
    b = _desc_maxima(d[n_all // 2:], ka)
    cands = [b[ka - 1], a[ka - 1]]
    for i in range(1, ka):
        cands.append(jnp.minimum(a[i - 1], b[ka - 1 - i]))
    tau = cands[0]
    for c in cands[1:]:
        tau = jnp.maximum(tau, c)
    m1 = jnp.maximum(a[0], b[0])
    mask = jnp.where(d >= tau, 1.0, 0.0)
    mask = jnp.where(d == m1, 0.0, mask)

    lp_full = _leaky(pts_ref[0])
    gsum = jnp.dot(lp_full, mask,
                   preferred_element_type=jnp.float32)

    p = ptile_ref[0]
    lp = _leaky(p)
    x0 = jnp.concatenate([lp, gsum], axis=0)
    acc = jnp.dot(wcat_ref[...], x0,
                  preferred_element_type=jnp.float32)
    h = (acc + bcat_ref[...]) * (1.0 / (k + 1.0)) + p

    for blk in range(nblk1):
        lph = _leaky(h)
        acc = jnp.dot(wf_ref[blk], lph,
                      preferred_element_type=jnp.float32)
        h = acc + bf_ref[blk] + h

    o_ref[0] = h.astype(o_ref.dtype)


def _run_chip(xyz, points, w_cat, b_cat, w_f, b_f):
    B, C, N = points.shape
    nblk1 = int(w_f.shape[0])
    if N % 1024 == 0:
        TN = 1024
    elif N % 512 == 0:
        TN = 512
    elif N % 128 == 0:
        TN = 128
    else:
        TN = N
    xyz_nc = jnp.transpose(xyz, (0, 2, 1))

    body = functools.partial(_fused_kernel, k=_K, nblk1=nblk1)
    return pl.pallas_call(
        body,
        out_shape=jax.ShapeDtypeStruct((B, C, N), points.dtype),
        grid=(B, N // TN),
        in_specs=[
            pl.BlockSpec((1, 3, TN), lambda b, n: (b, 0, n)),
            pl.BlockSpec((1, N, 3), lambda b, n: (b, 0, 0)),
            pl.BlockSpec((1, C, N), lambda b, n: (b, 0, 0)),
            pl.BlockSpec((1, C, TN), lambda b, n: (b, 0, n)),
            pl.BlockSpec((C, 2 * C), lambda b, n: (0, 0)),
            pl.BlockSpec((C, 1), lambda b, n: (0, 0)),
            pl.BlockSpec((nblk1, C, C), lambda b, n: (0, 0, 0)),
            pl.BlockSpec((nblk1, C, 1), lambda b, n: (0, 0, 0)),
        ],
        out_specs=pl.BlockSpec((1, C, TN), lambda b, n: (b, 0, n)),
        compiler_params=pltpu.CompilerParams(
            dimension_semantics=("parallel", "arbitrary")),
    )(xyz, xyz_nc, points, points, w_cat, b_cat, w_f, b_f)


def kernel(xyz, points, w_cat, b_cat, w_f, b_f):
    return _run_chip(xyz, points, w_cat, b_cat, w_f, b_f)


# speedup vs baseline: 1.0789x; 1.0789x over previous
import functools

import jax
import jax.numpy as jnp
from jax.experimental import pallas as pl
from jax.experimental.pallas import tpu as pltpu

_NEG_SLOPE = 0.01
_K = 16


def _leaky(x):
    return jnp.where(x > 0, x, _NEG_SLOPE * x)


def _fused_kernel(xq_ref, xall_ref, pts_ref, ptile_ref, wcat_ref, bcat_ref,
                  wf_ref, bf_ref, o_ref, *, k, nblk1):
    xall = xall_ref[0]
    xq = xq_ref[0]

    nj = jnp.sum(xall * xall, axis=1, keepdims=True)
    ni = jnp.sum(xq * xq, axis=0, keepdims=True)
    am2 = xall * -2.0
    a_hi = am2.astype(jnp.bfloat16)
    a_lo = (am2 - a_hi.astype(jnp.float32)).astype(jnp.bfloat16)
    q_hi = xq.astype(jnp.bfloat16)
    q_lo = (xq - q_hi.astype(jnp.float32)).astype(jnp.bfloat16)
    f32 = jnp.float32
    ip2 = (jnp.dot(a_hi, q_hi, preferred_element_type=f32)
           + (jnp.dot(a_hi, q_lo, preferred_element_type=f32)
              + jnp.dot(a_lo, q_hi, preferred_element_type=f32)))
    d = (ip2 + nj) + ni

    neg_inf = jnp.float32(-jnp.inf)
    n_all = d.shape[0]

    def _desc_maxima(dq, count):
        ms = [jnp.max(dq, axis=0, keepdims=True)]
        for _ in range(count - 1):
            ms.append(jnp.max(jnp.where(dq >= ms[-1], neg_inf, dq),
                              axis=0, keepdims=True))
        return ms

    ka = k + 1
    a = _desc_maxima(d[: n_all // 2], ka)
    b = _desc_maxima(d[n_all // 2:], ka)
    cands = [b[ka - 1], a[ka - 1]]
    for i in range(1, ka):
        cands.append(jnp.minimum(a[i - 1], b[ka - 1 - i]))
    tau = cands[0]
    for c in cands[1:]:
        tau = jnp.maximum(tau, c)
    m1 = jnp.maximum(a[0], b[0])
    mask = jnp.where(d >= tau, 1.0, 0.0)
    mask = jnp.where(d == m1, 0.0, mask)

    lp_full = _leaky(pts_ref[0])
    gsum = jnp.dot(lp_full, mask,
                   preferred_element_type=jnp.float32)

    p = ptile_ref[0]
    lp = _leaky(p)
    x0 = jnp.concatenate([lp, gsum], axis=0)
    acc = jnp.dot(wcat_ref[...], x0,
                  preferred_element_type=jnp.float32)
    h = (acc + bcat_ref[...]) * (1.0 / (k + 1.0)) + p

    for blk in range(nblk1):
        lph = _leaky(h)
        acc = jnp.dot(wf_ref[blk], lph,
                      preferred_element_type=jnp.float32)
        h = acc + bf_ref[blk] + h

    o_ref[0] = h.astype(o_ref.dtype)


def _run_chip(xyz, points, w_cat, b_cat, w_f, b_f):
    B, C, N = points.shape
    nblk1 = int(w_f.shape[0])
    if N % 1024 == 0:
        TN = 1024
    elif N % 512 == 0:
        TN = 512
    elif N % 128 == 0:
        TN = 128
    else:
        TN = N
    xyz_nc = jnp.transpose(xyz, (0, 2, 1))

    body = functools.partial(_fused_kernel, k=_K, nblk1=nblk1)
    return pl.pallas_call(
        body,
        out_shape=jax.ShapeDtypeStruct((B, C, N), points.dtype),
        grid=(B, N // TN),
        in_specs=[
            pl.BlockSpec((1, 3, TN), lambda b, n: (b, 0, n)),
            pl.BlockSpec((1, N, 3), lambda b, n: (b, 0, 0)),
            pl.BlockSpec((1, C, N), lambda b, n: (b, 0, 0)),
            pl.BlockSpec((1, C, TN), lambda b, n: (b, 0, n)),
            pl.BlockSpec((C, 2 * C), lambda b, n: (0, 0)),
            pl.BlockSpec((C, 1), lambda b, n: (0, 0)),
            pl.BlockSpec((nblk1, C, C), lambda b, n: (0, 0, 0)),
            pl.BlockSpec((nblk1, C, 1), lambda b, n: (0, 0, 0)),
        ],
        out_specs=pl.BlockSpec((1, C, TN), lambda b, n: (b, 0, n)),
        compiler_params=pltpu.CompilerParams(
            dimension_semantics=("parallel", "arbitrary")),
    )(xyz, xyz_nc, points, points, w_cat, b_cat, w_f, b_f)


def kernel(xyz, points, w_cat, b_cat, w_f, b_f):
    return _run_chip(xyz, points, w_cat, b_cat, w_f, b_f)


# speedup vs baseline: 1.3779x; 1.2772x over previous
import functools

import jax
import jax.numpy as jnp
from jax.experimental import pallas as pl
from jax.experimental.pallas import tpu as pltpu

_NEG_SLOPE = 0.01
_K = 16


def _leaky(x):
    return jnp.where(x > 0, x, _NEG_SLOPE * x)


def _fused_kernel(xq_ref, xall_ref, pts_ref, ptile_ref, wcat_ref, bcat_ref,
                  wf_ref, bf_ref, o_ref, *, k, nblk1):
    xall = xall_ref[0]
    xq = xq_ref[0]

    d = None
    for axis in range(3):
        diff = xall[:, axis:axis + 1] - xq[axis:axis + 1, :]
        sq = diff * diff
        d = sq if d is None else d + sq

    neg_inf = jnp.float32(-jnp.inf)
    n_all = d.shape[0]

    def _top4(v):
        r1 = v[0:8]
        fill = jnp.full_like(r1, neg_inf)
        r2, r3, r4 = fill, fill, fill
        for i in range(1, v.shape[0] // 8):
            s = v[8 * i:8 * (i + 1)]
            t1 = jnp.maximum(r1, s)
            b1 = jnp.minimum(r1, s)
            t2 = jnp.maximum(r2, b1)
            b2 = jnp.minimum(r2, b1)
            t3 = jnp.maximum(r3, b2)
            b3 = jnp.minimum(r3, b2)
            r4 = jnp.maximum(r4, b3)
            r1, r2, r3 = t1, t2, t3
        st = jnp.concatenate([r1, r2, r3, r4], axis=0)
        out = [jnp.max(st, axis=0, keepdims=True)]
        for _ in range(3):
            out.append(jnp.max(jnp.where(st >= out[-1], neg_inf, st),
                               axis=0, keepdims=True))
        return out

    def _desc_maxima(dq, count):
        if dq.shape[0] % 8 == 0 and dq.shape[0] >= 16 and count >= 4:
            ms = _top4(dq)
            while len(ms) <= count - 4:
                ms.extend(_top4(jnp.where(dq >= ms[-1], neg_inf, dq)))
            while len(ms) < count:
                ms.append(jnp.max(jnp.where(dq >= ms[-1], neg_inf, dq),
                                  axis=0, keepdims=True))
            return ms[:count]
        ms = [jnp.max(dq, axis=0, keepdims=True)]
        for _ in range(count - 1):
            ms.append(jnp.max(jnp.where(dq >= ms[-1], neg_inf, dq),
                              axis=0, keepdims=True))
        return ms

    ka = k + 1
    a = _desc_maxima(d[: n_all // 2], ka)
    b = _desc_maxima(d[n_all // 2:], ka)
    cands = [b[ka - 1], a[ka - 1]]
    for i in range(1, ka):
        cands.append(jnp.minimum(a[i - 1], b[ka - 1 - i]))
    tau = cands[0]
    for c in cands[1:]:
        tau = jnp.maximum(tau, c)
    m1 = jnp.maximum(a[0], b[0])
    mask = jnp.where((d >= tau) & (d != m1), 1.0, 0.0)

    lp_full = _leaky(pts_ref[0])
    gsum = jnp.dot(lp_full, mask,
                   preferred_element_type=jnp.float32)

    p = ptile_ref[0]
    lp = _leaky(p)
    x0 = jnp.concatenate([lp, gsum], axis=0)
    acc = jnp.dot(wcat_ref[...], x0,
                  preferred_element_type=jnp.float32)
    h = (acc + bcat_ref[...]) * (1.0 / (k + 1.0)) + p

    for blk in range(nblk1):
        lph = _leaky(h)
        acc = jnp.dot(wf_ref[blk], lph,
                      preferred_element_type=jnp.float32)
        h = acc + bf_ref[blk] + h

    o_ref[0] = h.astype(o_ref.dtype)


def _run_chip(xyz, points, w_cat, b_cat, w_f, b_f):
    B, C, N = points.shape
    nblk1 = int(w_f.shape[0])
    if N % 1024 == 0:
        TN = 1024
    elif N % 512 == 0:
        TN = 512
    elif N % 128 == 0:
        TN = 128
    else:
        TN = N
    xyz_nc = jnp.transpose(xyz, (0, 2, 1))

    body = functools.partial(_fused_kernel, k=_K, nblk1=nblk1)
    return pl.pallas_call(
        body,
        out_shape=jax.ShapeDtypeStruct((B, C, N), points.dtype),
        grid=(B, N // TN),
        in_specs=[
            pl.BlockSpec((1, 3, TN), lambda b, n: (b, 0, n)),
            pl.BlockSpec((1, N, 3), lambda b, n: (b, 0, 0)),
            pl.BlockSpec((1, C, N), lambda b, n: (b, 0, 0)),
            pl.BlockSpec((1, C, TN), lambda b, n: (b, 0, n)),
            pl.BlockSpec((C, 2 * C), lambda b, n: (0, 0)),
            pl.BlockSpec((C, 1), lambda b, n: (0, 0)),
            pl.BlockSpec((nblk1, C, C), lambda b, n: (0, 0, 0)),
            pl.BlockSpec((nblk1, C, 1), lambda b, n: (0, 0, 0)),
        ],
        out_specs=pl.BlockSpec((1, C, TN), lambda b, n: (b, 0, n)),
        compiler_params=pltpu.CompilerParams(
            dimension_semantics=("parallel", "arbitrary")),
    )(xyz, xyz_nc, points, points, w_cat, b_cat, w_f, b_f)


def kernel(xyz, points, w_cat, b_cat, w_f, b_f):
    return _run_chip(xyz, points, w_cat, b_cat, w_f, b_f)


# speedup vs baseline: 1.3883x; 1.0075x over previous
import functools

import jax
import jax.numpy as jnp
from jax.experimental import pallas as pl
from jax.experimental.pallas import tpu as pltpu

_NEG_SLOPE = 0.01
_K = 16


def _leaky(x):
    return jnp.where(x > 0, x, _NEG_SLOPE * x)


def _fused_kernel(xq_ref, xall_ref, pts_ref, ptile_ref, wcat_ref, bcat_ref,
                  wf_ref, bf_ref, o_ref, *, k, nblk1):
    xall = xall_ref[0]
    xq = xq_ref[0]

    d = None
    for axis in range(3):
        diff = xall[:, axis:axis + 1] - xq[axis:axis + 1, :]
        sq = diff * diff
        d = sq if d is None else d + sq

    neg_inf = jnp.float32(-jnp.inf)
    n_all = d.shape[0]

    def _top4(v, m_prev):
        def slab(i):
            s = v[8 * i:8 * (i + 1)]
            if m_prev is not None:
                s = jnp.where(s >= m_prev, neg_inf, s)
            return s

        r1 = slab(0)
        fill = jnp.full_like(r1, neg_inf)
        r2, r3, r4 = fill, fill, fill
        for i in range(1, v.shape[0] // 8):
            s = slab(i)
            t1 = jnp.maximum(r1, s)
            b1 = jnp.minimum(r1, s)
            t2 = jnp.maximum(r2, b1)
            b2 = jnp.minimum(r2, b1)
            t3 = jnp.maximum(r3, b2)
            b3 = jnp.minimum(r3, b2)
            r4 = jnp.maximum(r4, b3)
            r1, r2, r3 = t1, t2, t3
        st = jnp.concatenate([r1, r2, r3, r4], axis=0)
        out = [jnp.max(st, axis=0, keepdims=True)]
        for _ in range(3):
            out.append(jnp.max(jnp.where(st >= out[-1], neg_inf, st),
                               axis=0, keepdims=True))
        return out

    def _desc_maxima(dq, count):
        if dq.shape[0] % 8 == 0 and dq.shape[0] >= 16 and count >= 4:
            ms = _top4(dq, None)
            while len(ms) <= count - 4:
                ms.extend(_top4(dq, ms[-1]))
            while len(ms) < count:
                ms.append(jnp.max(jnp.where(dq >= ms[-1], neg_inf, dq),
                                  axis=0, keepdims=True))
            return ms[:count]
        ms = [jnp.max(dq, axis=0, keepdims=True)]
        for _ in range(count - 1):
            ms.append(jnp.max(jnp.where(dq >= ms[-1], neg_inf, dq),
                              axis=0, keepdims=True))
        return ms

    ka = k + 1
    a = _desc_maxima(d[: n_all // 2], ka)
    b = _desc_maxima(d[n_all // 2:], ka)
    cands = [b[ka - 1], a[ka - 1]]
    for i in range(1, ka):
        cands.append(jnp.minimum(a[i - 1], b[ka - 1 - i]))
    tau = cands[0]
    for c in cands[1:]:
        tau = jnp.maximum(tau, c)
    m1 = jnp.maximum(a[0], b[0])
    mask = jnp.where((d >= tau) & (d != m1), 1.0, 0.0)

    lp_full = _leaky(pts_ref[0])
    gsum = jnp.dot(lp_full, mask,
                   preferred_element_type=jnp.float32)

    p = ptile_ref[0]
    lp = _leaky(p)
    x0 = jnp.concatenate([lp, gsum], axis=0)
    acc = jnp.dot(wcat_ref[...], x0,
                  preferred_element_type=jnp.float32)
    h = (acc + bcat_ref[...]) * (1.0 / (k + 1.0)) + p

    for blk in range(nblk1):
        lph = _leaky(h)
        acc = jnp.dot(wf_ref[blk], lph,
                      preferred_element_type=jnp.float32)
        h = acc + bf_ref[blk] + h

    o_ref[0] = h.astype(o_ref.dtype)


def _run_chip(xyz, points, w_cat, b_cat, w_f, b_f):
    B, C, N = points.shape
    nblk1 = int(w_f.shape[0])
    if N % 1024 == 0:
        TN = 1024
    elif N % 512 == 0:
        TN = 512
    elif N % 128 == 0:
        TN = 128
    else:
        TN = N
    xyz_nc = jnp.transpose(xyz, (0, 2, 1))

    body = functools.partial(_fused_kernel, k=_K, nblk1=nblk1)
    return pl.pallas_call(
        body,
        out_shape=jax.ShapeDtypeStruct((B, C, N), points.dtype),
        grid=(B, N // TN),
        in_specs=[
            pl.BlockSpec((1, 3, TN), lambda b, n: (b, 0, n)),
            pl.BlockSpec((1, N, 3), lambda b, n: (b, 0, 0)),
            pl.BlockSpec((1, C, N), lambda b, n: (b, 0, 0)),
            pl.BlockSpec((1, C, TN), lambda b, n: (b, 0, n)),
            pl.BlockSpec((C, 2 * C), lambda b, n: (0, 0)),
            pl.BlockSpec((C, 1), lambda b, n: (0, 0)),
            pl.BlockSpec((nblk1, C, C), lambda b, n: (0, 0, 0)),
            pl.BlockSpec((nblk1, C, 1), lambda b, n: (0, 0, 0)),
        ],
        out_specs=pl.BlockSpec((1, C, TN), lambda b, n: (b, 0, n)),
        compiler_params=pltpu.CompilerParams(
            dimension_semantics=("parallel", "arbitrary")),
    )(xyz, xyz_nc, points, points, w_cat, b_cat, w_f, b_f)


def kernel(xyz, points, w_cat, b_cat, w_f, b_f):
    return _run_chip(xyz, points, w_cat, b_cat, w_f, b_f)


# speedup vs baseline: 1.3948x; 1.0047x over previous
import functools

import jax
import jax.numpy as jnp
from jax.experimental import pallas as pl
from jax.experimental.pallas import tpu as pltpu

_NEG_SLOPE = 0.01
_K = 16


def _leaky(x):
    return jnp.where(x > 0, x, _NEG_SLOPE * x)


def _fused_kernel(xq_ref, xall_ref, pts_ref, wcat_ref, bcat_ref,
                  wf_ref, bf_ref, o_ref, *, k, tn, nblk1):
    xall = xall_ref[0]
    xq = xq_ref[0]

    d = None
    for axis in range(3):
        diff = xall[:, axis:axis + 1] - xq[axis:axis + 1, :]
        sq = diff * diff
        d = sq if d is None else d + sq

    neg_inf = jnp.float32(-jnp.inf)
    n_all = d.shape[0]

    def _top4(v, m_prev):
        def slab(i):
            s = v[8 * i:8 * (i + 1)]
            if m_prev is not None:
                s = jnp.where(s >= m_prev, neg_inf, s)
            return s

        r1 = slab(0)
        fill = jnp.full_like(r1, neg_inf)
        r2, r3, r4 = fill, fill, fill
        for i in range(1, v.shape[0] // 8):
            s = slab(i)
            t1 = jnp.maximum(r1, s)
            b1 = jnp.minimum(r1, s)
            t2 = jnp.maximum(r2, b1)
            b2 = jnp.minimum(r2, b1)
            t3 = jnp.maximum(r3, b2)
            b3 = jnp.minimum(r3, b2)
            r4 = jnp.maximum(r4, b3)
            r1, r2, r3 = t1, t2, t3
        st = jnp.concatenate([r1, r2, r3, r4], axis=0)
        out = [jnp.max(st, axis=0, keepdims=True)]
        for _ in range(3):
            out.append(jnp.max(jnp.where(st >= out[-1], neg_inf, st),
                               axis=0, keepdims=True))
        return out

    def _desc_maxima(dq, count):
        if dq.shape[0] % 8 == 0 and dq.shape[0] >= 16 and count >= 4:
            ms = _top4(dq, None)
            while len(ms) <= count - 4:
                ms.extend(_top4(dq, ms[-1]))
            while len(ms) < count:
                ms.append(jnp.max(jnp.where(dq >= ms[-1], neg_inf, dq),
                                  axis=0, keepdims=True))
            return ms[:count]
        ms = [jnp.max(dq, axis=0, keepdims=True)]
        for _ in range(count - 1):
            ms.append(jnp.max(jnp.where(dq >= ms[-1], neg_inf, dq),
                              axis=0, keepdims=True))
        return ms

    ka = k + 1
    a = _desc_maxima(d[: n_all // 2], ka)
    b = _desc_maxima(d[n_all // 2:], ka)
    cands = [b[ka - 1], a[ka - 1]]
    for i in range(1, ka):
        cands.append(jnp.minimum(a[i - 1], b[ka - 1 - i]))
    tau = cands[0]
    for c in cands[1:]:
        tau = jnp.maximum(tau, c)
    m1 = jnp.maximum(a[0], b[0])
    mask = jnp.where((d >= tau) & (d != m1), 1.0, 0.0)

    lp_full = _leaky(pts_ref[0])
    gsum = jnp.dot(lp_full, mask,
                   preferred_element_type=jnp.float32)

    p = pts_ref[0, :, pl.ds(pl.program_id(1) * tn, tn)]
    lp = _leaky(p)
    x0 = jnp.concatenate([lp, gsum], axis=0)
    acc = jnp.dot(wcat_ref[...], x0,
                  preferred_element_type=jnp.float32)
    h = (acc + bcat_ref[...]) * (1.0 / (k + 1.0)) + p

    for blk in range(nblk1):
        lph = _leaky(h)
        acc = jnp.dot(wf_ref[blk], lph,
                      preferred_element_type=jnp.float32)
        h = acc + bf_ref[blk] + h

    o_ref[0] = h.astype(o_ref.dtype)


def _run_chip(xyz, points, w_cat, b_cat, w_f, b_f):
    B, C, N = points.shape
    nblk1 = int(w_f.shape[0])
    if N % 1024 == 0:
        TN = 1024
    elif N % 512 == 0:
        TN = 512
    elif N % 128 == 0:
        TN = 128
    else:
        TN = N
    xyz_nc = jnp.transpose(xyz, (0, 2, 1))

    body = functools.partial(_fused_kernel, k=_K, tn=TN, nblk1=nblk1)
    return pl.pallas_call(
        body,
        out_shape=jax.ShapeDtypeStruct((B, C, N), points.dtype),
        grid=(B, N // TN),
        in_specs=[
            pl.BlockSpec((1, 3, TN), lambda b, n: (b, 0, n)),
            pl.BlockSpec((1, N, 3), lambda b, n: (b, 0, 0)),
            pl.BlockSpec((1, C, N), lambda b, n: (b, 0, 0)),
            pl.BlockSpec((C, 2 * C), lambda b, n: (0, 0)),
            pl.BlockSpec((C, 1), lambda b, n: (0, 0)),
            pl.BlockSpec((nblk1, C, C), lambda b, n: (0, 0, 0)),
            pl.BlockSpec((nblk1, C, 1), lambda b, n: (0, 0, 0)),
        ],
        out_specs=pl.BlockSpec((1, C, TN), lambda b, n: (b, 0, n)),
        compiler_params=pltpu.CompilerParams(
            dimension_semantics=("parallel", "arbitrary")),
    )(xyz, xyz_nc, points, w_cat, b_cat, w_f, b_f)


def kernel(xyz, points, w_cat, b_cat, w_f, b_f):
    return _run_chip(xyz, points, w_cat, b_cat, w_f, b_f)
